# baseline (speedup 1.0000x reference)
import functools

import jax
import jax.numpy as jnp
from jax import lax
from jax.experimental import pallas as pl
from jax.experimental.pallas import tpu as pltpu
from jax.experimental.pallas import tpu_sc as plsc

NUM_CLASSES = 100000
LATENT_DIM = 64
BATCH = 16384

NC = 2
NS = 16
LANES = 16
NW = NC * NS
BPW = BATCH // NW
CHUNK = 128
NCHUNK = BPW // CHUNK
COLS = LATENT_DIM // LANES


def _make_partials():
    mesh = plsc.VectorSubcoreMesh(core_axis_name="c", subcore_axis_name="s")

    @functools.partial(
        pl.kernel,
        mesh=mesh,
        out_type=jax.ShapeDtypeStruct((NW, LANES), jnp.float32),
        compiler_params=pltpu.CompilerParams(use_tc_tiling_on_sc=False),
        scratch_types=[
            pltpu.VMEM((NCHUNK, CHUNK), jnp.int32),
            pltpu.VMEM((BPW, LATENT_DIM), jnp.float32),
            pltpu.VMEM((BPW, LATENT_DIM), jnp.float32),
            pltpu.VMEM((LANES,), jnp.float32),
            pltpu.SemaphoreType.DMA,
            pltpu.SemaphoreType.DMA,
        ],
    )
    def partials(emb_hbm, lab_hbm, cen_hbm, out_hbm,
                 idx_v, cbuf, ebuf, part_v, csem, esem):
        wid = lax.axis_index("s") * NC + lax.axis_index("c")
        base = wid * BPW

        for k in range(NCHUNK):
            pltpu.sync_copy(lab_hbm.at[pl.ds(base + k * CHUNK, CHUNK)],
                            idx_v.at[k])

        emb_cp = pltpu.async_copy(emb_hbm.at[pl.ds(base, BPW)], ebuf, esem)
        gathers = [
            pltpu.async_copy(cen_hbm.at[idx_v.at[k]],
                             cbuf.at[pl.ds(k * CHUNK, CHUNK)], csem)
            for k in range(NCHUNK)
        ]
        emb_cp.wait()
        for g in gathers:
            g.wait()

        def row_body(r, acc):
            for c in range(COLS):
                e = ebuf[r, pl.ds(c * LANES, LANES)]
                ce = cbuf[r, pl.ds(c * LANES, LANES)]
                d = e - ce
                acc = acc + d * d
            return acc

        acc = lax.fori_loop(0, BPW, row_body, jnp.zeros((LANES,), jnp.float32))
        part_v[...] = acc
        pltpu.sync_copy(part_v, out_hbm.at[wid])

    return partials


_partials_kernel = _make_partials()


def kernel(embeddings, labels, centers):
    labels = labels.astype(jnp.int32)
    parts = _partials_kernel(embeddings, labels, centers)
    return jnp.sum(parts) / embeddings.shape[0]


# speedup vs baseline: 1.8497x; 1.8497x over previous
import functools

import jax
import jax.numpy as jnp
from jax import lax
from jax.experimental import pallas as pl
from jax.experimental.pallas import tpu as pltpu
from jax.experimental.pallas import tpu_sc as plsc

NUM_CLASSES = 100000
LATENT_DIM = 64
BATCH = 16384

NC = 2
NS = 16
LANES = 16
NW = NC * NS
ROWS_PER_W = LATENT_DIM // NW
ECHUNK = 8192


def _make_partials():
    mesh = plsc.VectorSubcoreMesh(core_axis_name="c", subcore_axis_name="s")

    @functools.partial(
        pl.kernel,
        mesh=mesh,
        out_type=jax.ShapeDtypeStruct((NW, LANES), jnp.float32),
        compiler_params=pltpu.CompilerParams(use_tc_tiling_on_sc=True,
                                             needs_layout_passes=False),
        scratch_types=[
            pltpu.VMEM((BATCH,), jnp.int32),
            pltpu.VMEM((NUM_CLASSES,), jnp.float32),
            pltpu.VMEM((ECHUNK,), jnp.float32),
            pltpu.VMEM((LANES,), jnp.float32),
        ],
    )
    def partials(emb_hbm, lab_hbm, cen_hbm, out_hbm, lbuf, crow, ebuf, part_v):
        wid = lax.axis_index("s") * NC + lax.axis_index("c")
        pltpu.sync_copy(lab_hbm, lbuf)

        acc = jnp.zeros((LANES,), jnp.float32)
        for r in range(ROWS_PER_W):
            d = wid + NW * r
            pltpu.sync_copy(cen_hbm.at[d], crow)
            for cidx in range(BATCH // ECHUNK):
                pltpu.sync_copy(emb_hbm.at[d, pl.ds(cidx * ECHUNK, ECHUNK)],
                                ebuf)

                def body(j, acc, _b0=cidx * ECHUNK):
                    lab = lbuf[pl.ds(_b0 + j * LANES, LANES)]
                    e = ebuf[pl.ds(j * LANES, LANES)]
                    cg = plsc.load_gather(crow, [lab])
                    dlt = e - cg
                    return acc + dlt * dlt

                acc = lax.fori_loop(0, ECHUNK // LANES, body, acc)

        part_v[...] = acc
        pltpu.sync_copy(part_v, out_hbm.at[wid])

    return partials


_partials_kernel = _make_partials()


def kernel(embeddings, labels, centers):
    labels = labels.astype(jnp.int32)
    parts = _partials_kernel(embeddings.T, labels, centers.T)
    return jnp.sum(parts) / embeddings.shape[0]


# speedup vs baseline: 1.9935x; 1.0777x over previous
import functools

import jax
import jax.numpy as jnp
from jax import lax
from jax.experimental import pallas as pl
from jax.experimental.pallas import tpu as pltpu
from jax.experimental.pallas import tpu_sc as plsc

NUM_CLASSES = 100000
LATENT_DIM = 64
BATCH = 16384

NC = 2
NS = 16
LANES = 16
NW = NC * NS
ROWS_PER_W = LATENT_DIM // NW
ECHUNK = 8192
UNROLL = 4

CH = (NUM_CLASSES // 2 + 127) // 128 * 128
HALF_LO = (0, CH)
HALF_LEN = (CH, NUM_CLASSES - CH)


def _make_partials():
    mesh = plsc.VectorSubcoreMesh(core_axis_name="c", subcore_axis_name="s")

    @functools.partial(
        pl.kernel,
        mesh=mesh,
        out_type=jax.ShapeDtypeStruct((NW, LANES), jnp.float32),
        compiler_params=pltpu.CompilerParams(use_tc_tiling_on_sc=True,
                                             needs_layout_passes=False),
        scratch_types=[
            pltpu.VMEM((BATCH,), jnp.int32),
            pltpu.VMEM((CH,), jnp.float32),
            pltpu.VMEM((NUM_CLASSES - CH,), jnp.float32),
            pltpu.VMEM((ECHUNK,), jnp.float32),
            pltpu.VMEM((LANES,), jnp.float32),
            pltpu.SemaphoreType.DMA,
            pltpu.SemaphoreType.DMA,
        ],
    )
    def partials(emb_hbm, lab_hbm, cen_hbm, out_hbm,
                 lbuf, crow_a, crow_b, ebuf, part_v, sem0, sem1):
        wid = lax.axis_index("s") * NC + lax.axis_index("c")
        pltpu.sync_copy(lab_hbm, lbuf)

        items = [(r, h) for r in range(ROWS_PER_W) for h in range(2)]
        bufs = (crow_a, crow_b)
        sems = (sem0, sem1)

        def start(k):
            r, h = items[k]
            d = wid + NW * r
            return pltpu.async_copy(
                cen_hbm.at[d, pl.ds(HALF_LO[h], HALF_LEN[h])],
                bufs[h], sems[h])

        cp = start(0)
        acc = jnp.zeros((LANES,), jnp.float32)
        for k, (r, h) in enumerate(items):
            cp.wait()
            if k + 1 < len(items):
                cp = start(k + 1)
            d = wid + NW * r
            lo, ln, buf = HALF_LO[h], HALF_LEN[h], bufs[h]
            for cidx in range(BATCH // ECHUNK):
                pltpu.sync_copy(emb_hbm.at[d, pl.ds(cidx * ECHUNK, ECHUNK)],
                                ebuf)
                base = cidx * ECHUNK

                def body(j, a, _base=base, _lo=lo, _ln=ln, _buf=buf):
                    for u in range(UNROLL):
                        off = (j * UNROLL + u) * LANES
                        lab = lbuf[pl.ds(_base + off, LANES)]
                        e = ebuf[pl.ds(off, LANES)]
                        idx = lab - _lo
                        m = (idx >= 0) & (idx < _ln)
                        idxc = jnp.clip(idx, 0, _ln - 1)
                        g = plsc.load_gather(_buf, [idxc], mask=m)
                        dlt = jnp.where(m, e - g, 0.0)
                        a = a + dlt * dlt
                    return a

                acc = lax.fori_loop(0, ECHUNK // LANES // UNROLL, body, acc)

        part_v[...] = acc
        pltpu.sync_copy(part_v, out_hbm.at[wid])

    return partials


_partials_kernel = _make_partials()


def kernel(embeddings, labels, centers):
    labels = labels.astype(jnp.int32)
    parts = _partials_kernel(embeddings.T, labels, centers.T)
    return jnp.sum(parts) / embeddings.shape[0]


# speedup vs baseline: 2.0538x; 1.0302x over previous
import functools

import jax
import jax.numpy as jnp
from jax import lax
from jax.experimental import pallas as pl
from jax.experimental.pallas import tpu as pltpu
from jax.experimental.pallas import tpu_sc as plsc

NUM_CLASSES = 100000
LATENT_DIM = 64
BATCH = 16384

NC = 2
NS = 16
LANES = 16
NW = NC * NS
ROWS_PER_W = LATENT_DIM // NW
ECHUNK = 8192
UNROLL = 4

CH = (NUM_CLASSES // 2 + 127) // 128 * 128
HALF_LO = (0, CH)
HALF_LEN = (CH, NUM_CLASSES - CH)


def _make_partials():
    mesh = plsc.VectorSubcoreMesh(core_axis_name="c", subcore_axis_name="s")

    @functools.partial(
        pl.kernel,
        mesh=mesh,
        out_type=jax.ShapeDtypeStruct((NW, LANES), jnp.float32),
        compiler_params=pltpu.CompilerParams(use_tc_tiling_on_sc=True,
                                             needs_layout_passes=False),
        scratch_types=[
            pltpu.VMEM((BATCH,), jnp.int32),
            pltpu.VMEM((CH,), jnp.float32),
            pltpu.VMEM((NUM_CLASSES - CH,), jnp.float32),
            pltpu.VMEM((ECHUNK,), jnp.float32),
            pltpu.VMEM((LANES,), jnp.float32),
            pltpu.SemaphoreType.DMA,
            pltpu.SemaphoreType.DMA,
        ],
    )
    def partials(emb_hbm, lab_hbm, cen_hbm, out_hbm,
                 lbuf, crow_a, crow_b, ebuf, part_v, sem0, sem1):
        wid = lax.axis_index("s") * NC + lax.axis_index("c")
        pltpu.sync_copy(lab_hbm, lbuf)

        acc = jnp.zeros((LANES,), jnp.float32)
        for r in range(ROWS_PER_W):
            d = wid + NW * r
            cpa = pltpu.async_copy(cen_hbm.at[d, pl.ds(0, CH)], crow_a, sem0)
            cpb = pltpu.async_copy(
                cen_hbm.at[d, pl.ds(CH, NUM_CLASSES - CH)], crow_b, sem1)
            cpa.wait()
            cpb.wait()
            for cidx in range(BATCH // ECHUNK):
                pltpu.sync_copy(emb_hbm.at[d, pl.ds(cidx * ECHUNK, ECHUNK)],
                                ebuf)
                base = cidx * ECHUNK

                def body(j, a, _base=base):
                    for u in range(UNROLL):
                        off = (j * UNROLL + u) * LANES
                        lab = lbuf[pl.ds(_base + off, LANES)]
                        e = ebuf[pl.ds(off, LANES)]
                        in_a = lab < CH
                        ia = jnp.minimum(lab, CH - 1)
                        ib = jnp.maximum(lab - CH, 0)
                        ga = plsc.load_gather(crow_a, [ia], mask=in_a)
                        gb = plsc.load_gather(crow_b, [ib], mask=~in_a)
                        g = jnp.where(in_a, ga, gb)
                        dlt = e - g
                        a = a + dlt * dlt
                    return a

                acc = lax.fori_loop(0, ECHUNK // LANES // UNROLL, body, acc)

        part_v[...] = acc
        pltpu.sync_copy(part_v, out_hbm.at[wid])

    return partials


_partials_kernel = _make_partials()


def kernel(embeddings, labels, centers):
    labels = labels.astype(jnp.int32)
    parts = _partials_kernel(embeddings.T, labels, centers.T)
    return jnp.sum(parts) / embeddings.shape[0]


# speedup vs baseline: 2.0566x; 1.0014x over previous
import functools

import jax
import jax.numpy as jnp
from jax import lax
from jax.experimental import pallas as pl
from jax.experimental.pallas import tpu as pltpu
from jax.experimental.pallas import tpu_sc as plsc

NUM_CLASSES = 100000
LATENT_DIM = 64
BATCH = 16384

NC = 2
NS = 16
LANES = 16
NW = NC * NS
ROWS_PER_W = LATENT_DIM // NW
ECHUNK = 8192
UNROLL = 4

CH = (NUM_CLASSES // 2 + 127) // 128 * 128
HALF_LO = (0, CH)
HALF_LEN = (CH, NUM_CLASSES - CH)


def _make_partials():
    mesh = plsc.VectorSubcoreMesh(core_axis_name="c", subcore_axis_name="s")

    @functools.partial(
        pl.kernel,
        mesh=mesh,
        out_type=jax.ShapeDtypeStruct((NW, LANES), jnp.float32),
        compiler_params=pltpu.CompilerParams(use_tc_tiling_on_sc=True,
                                             needs_layout_passes=False,
                                             skip_device_barrier=True),
        scratch_types=[
            pltpu.VMEM((BATCH,), jnp.int32),
            pltpu.VMEM((CH,), jnp.float32),
            pltpu.VMEM((NUM_CLASSES - CH,), jnp.float32),
            pltpu.VMEM((ECHUNK,), jnp.float32),
            pltpu.VMEM((LANES,), jnp.float32),
            pltpu.SemaphoreType.DMA,
            pltpu.SemaphoreType.DMA,
        ],
    )
    def partials(emb_hbm, lab_hbm, cen_hbm, out_hbm,
                 lbuf, crow_a, crow_b, ebuf, part_v, sem0, sem1):
        wid = lax.axis_index("s") * NC + lax.axis_index("c")
        pltpu.sync_copy(lab_hbm, lbuf)

        acc = jnp.zeros((LANES,), jnp.float32)
        for r in range(ROWS_PER_W):
            d = wid + NW * r
            cpa = pltpu.async_copy(cen_hbm.at[d, pl.ds(0, CH)], crow_a, sem0)
            cpb = pltpu.async_copy(
                cen_hbm.at[d, pl.ds(CH, NUM_CLASSES - CH)], crow_b, sem1)
            cpa.wait()
            cpb.wait()
            for cidx in range(BATCH // ECHUNK):
                pltpu.sync_copy(emb_hbm.at[d, pl.ds(cidx * ECHUNK, ECHUNK)],
                                ebuf)
                base = cidx * ECHUNK

                def body(j, a, _base=base):
                    for u in range(UNROLL):
                        off = (j * UNROLL + u) * LANES
                        lab = lbuf[pl.ds(_base + off, LANES)]
                        e = ebuf[pl.ds(off, LANES)]
                        in_a = lab < CH
                        ia = jnp.minimum(lab, CH - 1)
                        ib = jnp.maximum(lab - CH, 0)
                        ga = plsc.load_gather(crow_a, [ia], mask=in_a)
                        gb = plsc.load_gather(crow_b, [ib], mask=~in_a)
                        g = jnp.where(in_a, ga, gb)
                        dlt = e - g
                        a = a + dlt * dlt
                    return a

                acc = lax.fori_loop(0, ECHUNK // LANES // UNROLL, body, acc)

        part_v[...] = acc
        pltpu.sync_copy(part_v, out_hbm.at[wid])

    return partials


_partials_kernel = _make_partials()


def kernel(embeddings, labels, centers):
    labels = labels.astype(jnp.int32)
    parts = _partials_kernel(embeddings.T, labels, centers.T)
    return jnp.sum(parts) / embeddings.shape[0]


# speedup vs baseline: 2.3202x; 1.1282x over previous
import functools

import jax
import jax.numpy as jnp
from jax import lax
from jax.experimental import pallas as pl
from jax.experimental.pallas import tpu as pltpu
from jax.experimental.pallas import tpu_sc as plsc

NUM_CLASSES = 100000
LATENT_DIM = 64
BATCH = 16384

NC = 2
NS = 16
LANES = 16
NW = NC * NS
ROWS_PER_W = LATENT_DIM // NW
ECHUNK = 4096
UNROLL = 4

CH = (NUM_CLASSES // 2 + 127) // 128 * 128
HALF_LO = (0, CH)
HALF_LEN = (CH, NUM_CLASSES - CH)


def _make_partials():
    mesh = plsc.VectorSubcoreMesh(core_axis_name="c", subcore_axis_name="s")

    @functools.partial(
        pl.kernel,
        mesh=mesh,
        out_type=jax.ShapeDtypeStruct((NW, LANES), jnp.float32),
        compiler_params=pltpu.CompilerParams(use_tc_tiling_on_sc=True,
                                             needs_layout_passes=False,
                                             skip_device_barrier=True),
        scratch_types=[
            pltpu.VMEM((BATCH,), jnp.int32),
            pltpu.VMEM((CH,), jnp.float32),
            pltpu.VMEM((NUM_CLASSES - CH,), jnp.float32),
            pltpu.VMEM((ECHUNK,), jnp.float32),
            pltpu.VMEM((ECHUNK,), jnp.float32),
            pltpu.VMEM((LANES,), jnp.float32),
            pltpu.SemaphoreType.DMA,
            pltpu.SemaphoreType.DMA,
            pltpu.SemaphoreType.DMA,
            pltpu.SemaphoreType.DMA,
        ],
    )
    def partials(emb_hbm, lab_hbm, cen_hbm, out_hbm,
                 lbuf, crow_a, crow_b, ebuf0, ebuf1, part_v,
                 sem0, sem1, esem0, esem1):
        wid = lax.axis_index("s") * NC + lax.axis_index("c")

        NCHUNKS = BATCH // ECHUNK
        ebufs = (ebuf0, ebuf1)
        esems = (esem0, esem1)

        def start_crow(r):
            d = wid + NW * r
            return (pltpu.async_copy(cen_hbm.at[d, pl.ds(0, CH)], crow_a,
                                     sem0),
                    pltpu.async_copy(
                        cen_hbm.at[d, pl.ds(CH, NUM_CLASSES - CH)], crow_b,
                        sem1))

        def start_e(r, cidx):
            d = wid + NW * r
            return pltpu.async_copy(
                emb_hbm.at[d, pl.ds(cidx * ECHUNK, ECHUNK)],
                ebufs[cidx % 2], esems[cidx % 2])

        cps = start_crow(0)
        ecp = start_e(0, 0)
        pltpu.sync_copy(lab_hbm, lbuf)

        acc = jnp.zeros((LANES,), jnp.float32)
        for r in range(ROWS_PER_W):
            cps[0].wait()
            cps[1].wait()
            for cidx in range(NCHUNKS):
                ecp.wait()
                if cidx + 1 < NCHUNKS:
                    ecp = start_e(r, cidx + 1)
                base = cidx * ECHUNK
                ebuf_r = ebufs[cidx % 2]

                def body(j, a, _base=base, _ebuf=ebuf_r):
                    for u in range(UNROLL):
                        off = (j * UNROLL + u) * LANES
                        lab = lbuf[pl.ds(_base + off, LANES)]
                        e = _ebuf[pl.ds(off, LANES)]
                        in_a = lab < CH
                        ia = jnp.minimum(lab, CH - 1)
                        ib = jnp.maximum(lab - CH, 0)
                        ga = plsc.load_gather(crow_a, [ia], mask=in_a)
                        gb = plsc.load_gather(crow_b, [ib], mask=~in_a)
                        g = jnp.where(in_a, ga, gb)
                        dlt = e - g
                        a = a + dlt * dlt
                    return a

                acc = lax.fori_loop(0, ECHUNK // LANES // UNROLL, body, acc)
            if r + 1 < ROWS_PER_W:
                cps = start_crow(r + 1)
                ecp = start_e(r + 1, 0)

        part_v[...] = acc
        pltpu.sync_copy(part_v, out_hbm.at[wid])

    return partials


_partials_kernel = _make_partials()


def kernel(embeddings, labels, centers):
    labels = labels.astype(jnp.int32)
    parts = _partials_kernel(embeddings.T, labels, centers.T)
    return jnp.sum(parts) / embeddings.shape[0]
